# baseline (speedup 1.0000x reference)
import jax
import jax.numpy as jnp
from jax import lax
from jax.experimental import pallas as pl
from jax.experimental.pallas import tpu as pltpu
from jax.experimental.pallas import tpu_sc as plsc

N_VERT = 100000
N_F0, N_E0, N_F1, N_E1 = 50000, 16, 20000, 2
P0, P1 = N_F0 // 2, N_F1 // 2

NC, NS = 2, 16
NW = NC * NS
L = 16
CHUNK = 1024


def _shard(n):
    s = -(-n // NW)
    return -(-s // L) * L


SP0 = _shard(P0)
SP1 = _shard(P1)


def _chunks(total):
    return [(st, min(CHUNK, total - st)) for st in range(0, total, CHUNK)]


def _fill_positions(pos_v, base, size, parity):
    lane2 = lax.iota(jnp.int32, L) * 2
    start = base * 2 + parity
    for v in range(size // L):
        pos_v[pl.ds(v * L, L)] = lane2 + (start + v * 2 * L)


def _gather_body(table, f0, e0, f1, e1,
                 o_b0, o_d0, o_e0, o_b1, o_d1, o_e1,
                 pb0_v, pd0_v, ib0_v, id0_v, b0_v, d0_v,
                 pb1_v, pd1_v, ib1_v, id1_v, b1_v, d1_v,
                 idxe0_v, rowse0_v, idxe1_v, rowse1_v,
                 si0, si1, si2, si3, sem, sem_e):
    wid = lax.axis_index("s") * NC + lax.axis_index("c")

    off0 = jnp.minimum(wid * SP0, P0 - SP0)
    off1 = jnp.minimum(wid * SP1, P1 - SP1)

    _fill_positions(pb0_v, off0, SP0, 0)
    _fill_positions(pd0_v, off0, SP0, 1)
    _fill_positions(pb1_v, off1, SP1, 0)
    _fill_positions(pd1_v, off1, SP1, 1)

    lists = ((f0, pb0_v, ib0_v, b0_v, SP0),
             (f1, pb1_v, ib1_v, b1_v, SP1),
             (f0, pd0_v, id0_v, d0_v, SP0),
             (f1, pd1_v, id1_v, d1_v, SP1))
    stage1 = [pltpu.async_copy(src.at[pos_v], idx_v, si)
              for (src, pos_v, idx_v, _, _), si in zip(lists, (si0, si1, si2, si3))]
    copies = []
    for (_, _, idx_v, rows_v, _), c1 in zip(lists, stage1):
        c1.wait()
        copies.append(pltpu.async_copy(table.at[idx_v], rows_v, sem))

    @pl.when(wid == 0)
    def _essentials():
        pltpu.sync_copy(e0, idxe0_v)
        pltpu.sync_copy(e1, idxe1_v)
        pltpu.async_copy(table.at[idxe0_v], rowse0_v, sem_e).wait()
        pltpu.async_copy(table.at[idxe1_v], rowse1_v, sem_e).wait()
        pltpu.sync_copy(rowse0_v, o_e0)
        pltpu.sync_copy(rowse1_v, o_e1)

    for c in copies:
        c.wait()

    pltpu.sync_copy(b0_v, o_b0.at[pl.ds(off0, SP0)])
    pltpu.sync_copy(d0_v, o_d0.at[pl.ds(off0, SP0)])
    pltpu.sync_copy(b1_v, o_b1.at[pl.ds(off1, SP1)])
    pltpu.sync_copy(d1_v, o_d1.at[pl.ds(off1, SP1)])


@jax.jit
def kernel(filtration, finite_idx_0, essential_idx_0, finite_idx_1, essential_idx_1):
    b0, d0, e0, b1, d1, e1 = pl.kernel(
        _gather_body,
        out_type=(
            jax.ShapeDtypeStruct((P0,), jnp.float32),
            jax.ShapeDtypeStruct((P0,), jnp.float32),
            jax.ShapeDtypeStruct((N_E0,), jnp.float32),
            jax.ShapeDtypeStruct((P1,), jnp.float32),
            jax.ShapeDtypeStruct((P1,), jnp.float32),
            jax.ShapeDtypeStruct((N_E1,), jnp.float32),
        ),
        mesh=plsc.VectorSubcoreMesh(
            core_axis_name="c", subcore_axis_name="s", num_cores=NC, num_subcores=NS
        ),
        scratch_types=[
            pltpu.VMEM((SP0,), jnp.int32),
            pltpu.VMEM((SP0,), jnp.int32),
            pltpu.VMEM((SP0,), jnp.int32),
            pltpu.VMEM((SP0,), jnp.int32),
            pltpu.VMEM((SP0,), jnp.float32),
            pltpu.VMEM((SP0,), jnp.float32),
            pltpu.VMEM((SP1,), jnp.int32),
            pltpu.VMEM((SP1,), jnp.int32),
            pltpu.VMEM((SP1,), jnp.int32),
            pltpu.VMEM((SP1,), jnp.int32),
            pltpu.VMEM((SP1,), jnp.float32),
            pltpu.VMEM((SP1,), jnp.float32),
            pltpu.VMEM((N_E0,), jnp.int32),
            pltpu.VMEM((N_E0,), jnp.float32),
            pltpu.VMEM((N_E1,), jnp.int32),
            pltpu.VMEM((N_E1,), jnp.float32),
            pltpu.SemaphoreType.DMA,
            pltpu.SemaphoreType.DMA,
            pltpu.SemaphoreType.DMA,
            pltpu.SemaphoreType.DMA,
            pltpu.SemaphoreType.DMA,
            pltpu.SemaphoreType.DMA,
        ],
    )(filtration, finite_idx_0, essential_idx_0, finite_idx_1, essential_idx_1)

    return (
        jnp.stack([b0, d0], axis=1),
        e0.reshape(-1, 1),
        jnp.stack([b1, d1], axis=1),
        e1.reshape(-1, 1),
    )


# speedup vs baseline: 1.0291x; 1.0291x over previous
import jax
import jax.numpy as jnp
from jax import lax
from jax.experimental import pallas as pl
from jax.experimental.pallas import tpu as pltpu
from jax.experimental.pallas import tpu_sc as plsc

N_VERT = 100000
N_F0, N_E0, N_F1, N_E1 = 50000, 16, 20000, 2
P0, P1 = N_F0 // 2, N_F1 // 2

NC, NS = 2, 16
NW = NC * NS
L = 16
CHUNK = 1024


def _shard(n):
    s = -(-n // NW)
    return -(-s // L) * L


SP0 = _shard(P0)
SP1 = _shard(P1)


def _chunks(total):
    return [(st, min(CHUNK, total - st)) for st in range(0, total, CHUNK)]


def _fill_positions(pos_v, base, size, parity):
    lane2 = lax.iota(jnp.int32, L) * 2
    start = base * 2 + parity
    for v in range(size // L):
        pos_v[pl.ds(v * L, L)] = lane2 + (start + v * 2 * L)


def _gather_body(table, f0, e0, f1, e1,
                 o_b0, o_d0, o_e0, o_b1, o_d1, o_e1,
                 pb0_v, pd0_v, ib0_v, id0_v, b0_v, d0_v,
                 pb1_v, pd1_v, ib1_v, id1_v, b1_v, d1_v,
                 idxe0_v, rowse0_v, idxe1_v, rowse1_v, sem, sem_e):
    wid = lax.axis_index("s") * NC + lax.axis_index("c")

    off0 = jnp.minimum(wid * SP0, P0 - SP0)
    off1 = jnp.minimum(wid * SP1, P1 - SP1)

    _fill_positions(pb0_v, off0, SP0, 0)
    _fill_positions(pd0_v, off0, SP0, 1)
    _fill_positions(pb1_v, off1, SP1, 0)
    _fill_positions(pd1_v, off1, SP1, 1)

    lists = ((f0, pb0_v, ib0_v, b0_v, SP0),
             (f1, pb1_v, ib1_v, b1_v, SP1),
             (f0, pd0_v, id0_v, d0_v, SP0),
             (f1, pd1_v, id1_v, d1_v, SP1))
    stage1 = [pltpu.async_copy(src.at[pos_v], idx_v, sem)
              for src, pos_v, idx_v, _, _ in lists]
    for c in stage1:
        c.wait()
    copies = [pltpu.async_copy(table.at[idx_v], rows_v, sem)
              for _, _, idx_v, rows_v, _ in lists]

    @pl.when(wid == 0)
    def _essentials():
        pltpu.sync_copy(e0, idxe0_v)
        pltpu.sync_copy(e1, idxe1_v)
        pltpu.async_copy(table.at[idxe0_v], rowse0_v, sem_e).wait()
        pltpu.async_copy(table.at[idxe1_v], rowse1_v, sem_e).wait()
        pltpu.sync_copy(rowse0_v, o_e0)
        pltpu.sync_copy(rowse1_v, o_e1)

    for c in copies:
        c.wait()

    pltpu.sync_copy(b0_v, o_b0.at[pl.ds(off0, SP0)])
    pltpu.sync_copy(d0_v, o_d0.at[pl.ds(off0, SP0)])
    pltpu.sync_copy(b1_v, o_b1.at[pl.ds(off1, SP1)])
    pltpu.sync_copy(d1_v, o_d1.at[pl.ds(off1, SP1)])


@jax.jit
def kernel(filtration, finite_idx_0, essential_idx_0, finite_idx_1, essential_idx_1):
    b0, d0, e0, b1, d1, e1 = pl.kernel(
        _gather_body,
        out_type=(
            jax.ShapeDtypeStruct((P0,), jnp.float32),
            jax.ShapeDtypeStruct((P0,), jnp.float32),
            jax.ShapeDtypeStruct((N_E0,), jnp.float32),
            jax.ShapeDtypeStruct((P1,), jnp.float32),
            jax.ShapeDtypeStruct((P1,), jnp.float32),
            jax.ShapeDtypeStruct((N_E1,), jnp.float32),
        ),
        mesh=plsc.VectorSubcoreMesh(
            core_axis_name="c", subcore_axis_name="s", num_cores=NC, num_subcores=NS
        ),
        scratch_types=[
            pltpu.VMEM((SP0,), jnp.int32),
            pltpu.VMEM((SP0,), jnp.int32),
            pltpu.VMEM((SP0,), jnp.int32),
            pltpu.VMEM((SP0,), jnp.int32),
            pltpu.VMEM((SP0,), jnp.float32),
            pltpu.VMEM((SP0,), jnp.float32),
            pltpu.VMEM((SP1,), jnp.int32),
            pltpu.VMEM((SP1,), jnp.int32),
            pltpu.VMEM((SP1,), jnp.int32),
            pltpu.VMEM((SP1,), jnp.int32),
            pltpu.VMEM((SP1,), jnp.float32),
            pltpu.VMEM((SP1,), jnp.float32),
            pltpu.VMEM((N_E0,), jnp.int32),
            pltpu.VMEM((N_E0,), jnp.float32),
            pltpu.VMEM((N_E1,), jnp.int32),
            pltpu.VMEM((N_E1,), jnp.float32),
            pltpu.SemaphoreType.DMA,
            pltpu.SemaphoreType.DMA,
        ],
    )(filtration, finite_idx_0, essential_idx_0, finite_idx_1, essential_idx_1)

    return (
        jnp.stack([b0, d0], axis=1),
        e0.reshape(-1, 1),
        jnp.stack([b1, d1], axis=1),
        e1.reshape(-1, 1),
    )
